# Initial kernel scaffold; baseline (speedup 1.0000x reference)
import jax
import jax.numpy as jnp
from jax.experimental import pallas as pl


def kernel(tgt_index, knn_dists, nmt_prob, W1, b1, W2, b2):
    raise NotImplementedError("write your pallas kernel here")


# speedup vs baseline: 1.7452x; 1.7452x over previous
import functools

import jax
import jax.numpy as jnp
from jax import lax
from jax.experimental import pallas as pl
from jax.experimental.pallas import tpu as pltpu
from jax.experimental.pallas import tpu_sc as plsc


def _tc_body(idx_ref, d_ref, w1_ref, b1_ref, w2t_ref, b2_ref, out_ref):
    idx = idx_ref[...]
    d = d_ref[...]
    R, K = idx.shape
    eq = idx[:, :, None] == idx[:, None, :]
    r0 = lax.broadcasted_iota(jnp.int32, (K, K), 0)
    r1 = lax.broadcasted_iota(jnp.int32, (K, K), 1)
    seen = jnp.sum(jnp.where(eq & (r1 < r0)[None], 1, 0), axis=-1) > 0
    is_new = ((idx != 0) & ~seen).astype(jnp.float32)
    tri = (r0 <= r1).astype(jnp.float32)
    counts = jnp.dot(is_new, tri, precision=lax.Precision.HIGHEST)
    feat = jnp.concatenate([d, counts], axis=-1)
    h = jnp.tanh(
        jnp.dot(feat, w1_ref[...], precision=lax.Precision.HIGHEST)
        + b1_ref[...]
    )
    logit = jnp.sum(h * w2t_ref[...], axis=-1, keepdims=True) + b2_ref[...]
    tempe = jax.nn.sigmoid(logit)
    x = -d * tempe
    x = x - jnp.max(x, axis=-1, keepdims=True)
    e = jnp.exp(x)
    p = e / jnp.sum(e, axis=-1, keepdims=True)
    comb = jnp.sum(eq.astype(jnp.float32) * p[:, None, :], axis=-1)
    out_ref[...] = comb


def _tc_combine(idx, d, W1, b1, W2, b2):
    R, K = idx.shape
    return pl.pallas_call(
        _tc_body,
        out_shape=jax.ShapeDtypeStruct((R, K), jnp.float32),
    )(idx, d, W1, b1.reshape(1, -1), W2.reshape(1, -1), b2.reshape(1, 1))


@functools.cache
def _make_sc_scatter(R, K, V):
    NC, NS = 2, 16
    NW = NC * NS
    RPW = R // NW
    assert R % NW == 0 and K % 16 == 0 and V % 8 == 0
    mesh = plsc.VectorSubcoreMesh(core_axis_name="c", subcore_axis_name="s")

    @functools.partial(
        pl.kernel,
        mesh=mesh,
        out_type=jax.ShapeDtypeStruct((R * V,), jnp.float32),
        scratch_types=[
            pltpu.VMEM((V,), jnp.float32),
            pltpu.VMEM((RPW, K), jnp.int32),
            pltpu.VMEM((RPW, K), jnp.float32),
            pltpu.SemaphoreType.DMA,
            pltpu.SemaphoreType.DMA,
        ],
    )
    def sc_scatter(zeros_hbm, idx_hbm, val_hbm, out_hbm,
                   zero_v, idx_v, val_v, sem_z, sem_s):
        wid = lax.axis_index("s") * NC + lax.axis_index("c")
        base_row = wid * RPW
        pltpu.sync_copy(zeros_hbm, zero_v)
        pltpu.sync_copy(idx_hbm.at[pl.ds(base_row, RPW)], idx_v)
        pltpu.sync_copy(val_hbm.at[pl.ds(base_row, RPW)], val_v)
        zcopies = [
            pltpu.async_copy(
                zero_v, out_hbm.at[pl.ds((base_row + r) * V, V)], sem_z)
            for r in range(RPW)
        ]
        for r in range(RPW):
            row_off = (base_row + r) * V
            for c in range(K // 16):
                sl = (r, pl.ds(c * 16, 16))
                idx_v[sl] = idx_v[sl] + row_off
        for cp in zcopies:
            cp.wait()
        scopies = [
            pltpu.async_copy(val_v.at[r], out_hbm.at[idx_v.at[r]], sem_s)
            for r in range(RPW)
        ]
        for cp in scopies:
            cp.wait()

    return sc_scatter


def kernel(tgt_index, knn_dists, nmt_prob, W1, b1, W2, b2):
    B, S, K = knn_dists.shape
    V = nmt_prob.shape[-1]
    R = B * S
    idx = tgt_index.reshape(R, K).astype(jnp.int32)
    d = knn_dists.reshape(R, K).astype(jnp.float32)
    vals = _tc_combine(idx, d, W1, b1, W2, b2)
    zeros_src = jnp.zeros((V,), jnp.float32)
    out_flat = _make_sc_scatter(R, K, V)(zeros_src, idx, vals)
    return out_flat.reshape(B, S, V)


# speedup vs baseline: 5.0995x; 2.9220x over previous
import functools

import jax
import jax.numpy as jnp
from jax import lax
from jax.experimental import pallas as pl
from jax.experimental.pallas import tpu as pltpu
from jax.experimental.pallas import tpu_sc as plsc


def _tc_body(idx_ref, d_ref, w1_ref, b1_ref, w2t_ref, b2_ref, out_ref):
    idx = idx_ref[...]
    d = d_ref[...]
    R, K = idx.shape
    eq = idx[:, :, None] == idx[:, None, :]
    r0 = lax.broadcasted_iota(jnp.int32, (K, K), 0)
    r1 = lax.broadcasted_iota(jnp.int32, (K, K), 1)
    seen = jnp.sum(jnp.where(eq & (r1 < r0)[None], 1, 0), axis=-1) > 0
    is_new = ((idx != 0) & ~seen).astype(jnp.float32)
    tri = (r0 <= r1).astype(jnp.float32)
    counts = jnp.dot(is_new, tri, precision=lax.Precision.HIGHEST)
    feat = jnp.concatenate([d, counts], axis=-1)
    h = jnp.tanh(
        jnp.dot(feat, w1_ref[...], precision=lax.Precision.HIGHEST)
        + b1_ref[...]
    )
    logit = jnp.sum(h * w2t_ref[...], axis=-1, keepdims=True) + b2_ref[...]
    tempe = jax.nn.sigmoid(logit)
    x = -d * tempe
    x = x - jnp.max(x, axis=-1, keepdims=True)
    e = jnp.exp(x)
    p = e / jnp.sum(e, axis=-1, keepdims=True)
    comb = jnp.sum(eq.astype(jnp.float32) * p[:, None, :], axis=-1)
    out_ref[...] = comb


def _tc_combine(idx, d, W1, b1, W2, b2):
    R, K = idx.shape
    return pl.pallas_call(
        _tc_body,
        out_shape=jax.ShapeDtypeStruct((R, K), jnp.float32),
    )(idx, d, W1, b1.reshape(1, -1), W2.reshape(1, -1), b2.reshape(1, 1))


@functools.cache
def _make_sc_scatter(B, S, K, V):
    NC, NS = 2, 16
    NW = NC * NS
    assert B == NW and K % 16 == 0
    CW = 12288
    NCHUNK = V // CW
    TW = V - NCHUNK * CW
    mesh = plsc.VectorSubcoreMesh(core_axis_name="c", subcore_axis_name="s")

    def _scatter_halves(buf, idx_v, val_v, base, width, vals_are_zero):
        for s in range(S):
            srow = jnp.full((16,), s, jnp.int32)
            for h in range(K // 16):
                iv = idx_v[s, pl.ds(h * 16, 16)]
                m = (iv >= base) & (iv < base + width)
                loc = jnp.where(m, iv - base, 0)
                if vals_are_zero:
                    vv = jnp.zeros((16,), jnp.float32)
                else:
                    vv = val_v[s, pl.ds(h * 16, 16)]
                plsc.store_scatter(buf, [srow, loc], vv, mask=m)

    @functools.partial(
        pl.kernel,
        mesh=mesh,
        out_type=jax.ShapeDtypeStruct((B, S, V), jnp.float32),
        compiler_params=pltpu.CompilerParams(needs_layout_passes=False),
        scratch_types=[
            pltpu.VMEM((S, CW), jnp.float32),
            pltpu.VMEM((S, TW), jnp.float32),
            pltpu.VMEM((S, K), jnp.int32),
            pltpu.VMEM((S, K), jnp.float32),
        ],
    )
    def sc_scatter(zeros_hbm, idx_hbm, val_hbm, out_hbm,
                   buf, tailbuf, idx_v, val_v):
        b = lax.axis_index("s") * NC + lax.axis_index("c")
        pltpu.sync_copy(zeros_hbm.at[:, pl.ds(0, CW)], buf)
        pltpu.sync_copy(zeros_hbm.at[:, pl.ds(CW, TW)], tailbuf)
        pltpu.sync_copy(idx_hbm.at[pl.ds(b * S, S)], idx_v)
        pltpu.sync_copy(val_hbm.at[pl.ds(b * S, S)], val_v)
        for c in range(NCHUNK):
            base = c * CW
            _scatter_halves(buf, idx_v, val_v, base, CW, False)
            pltpu.sync_copy(buf, out_hbm.at[b, :, pl.ds(base, CW)])
            if c + 1 < NCHUNK:
                _scatter_halves(buf, idx_v, val_v, base, CW, True)
        base = NCHUNK * CW
        _scatter_halves(tailbuf, idx_v, val_v, base, TW, False)
        pltpu.sync_copy(tailbuf, out_hbm.at[b, :, pl.ds(base, TW)])

    return sc_scatter


def kernel(tgt_index, knn_dists, nmt_prob, W1, b1, W2, b2):
    B, S, K = knn_dists.shape
    V = nmt_prob.shape[-1]
    R = B * S
    idx = tgt_index.reshape(R, K).astype(jnp.int32)
    d = knn_dists.reshape(R, K).astype(jnp.float32)
    vals = _tc_combine(idx, d, W1, b1, W2, b2)
    CW, TW = 12288, V - (V // 12288) * 12288
    zeros_src = jnp.zeros((S, CW + TW), jnp.float32)
    return _make_sc_scatter(B, S, K, V)(zeros_src, idx, vals)


# speedup vs baseline: 5.1821x; 1.0162x over previous
import functools

import jax
import jax.numpy as jnp
from jax import lax
from jax.experimental import pallas as pl
from jax.experimental.pallas import tpu as pltpu
from jax.experimental.pallas import tpu_sc as plsc


def _tc_body(idx_ref, d_ref, w1_ref, b1_ref, w2t_ref, b2_ref, out_ref):
    idx = idx_ref[...]
    d = d_ref[...]
    R, K = idx.shape
    eq = idx[:, :, None] == idx[:, None, :]
    r0 = lax.broadcasted_iota(jnp.int32, (K, K), 0)
    r1 = lax.broadcasted_iota(jnp.int32, (K, K), 1)
    seen = jnp.sum(jnp.where(eq & (r1 < r0)[None], 1, 0), axis=-1) > 0
    is_new = ((idx != 0) & ~seen).astype(jnp.float32)
    tri = (r0 <= r1).astype(jnp.float32)
    counts = jnp.dot(is_new, tri, precision=lax.Precision.HIGHEST)
    feat = jnp.concatenate([d, counts], axis=-1)
    h = jnp.tanh(
        jnp.dot(feat, w1_ref[...], precision=lax.Precision.HIGHEST)
        + b1_ref[...]
    )
    logit = jnp.sum(h * w2t_ref[...], axis=-1, keepdims=True) + b2_ref[...]
    tempe = jax.nn.sigmoid(logit)
    x = -d * tempe
    x = x - jnp.max(x, axis=-1, keepdims=True)
    e = jnp.exp(x)
    p = e / jnp.sum(e, axis=-1, keepdims=True)
    comb = jnp.sum(eq.astype(jnp.float32) * p[:, None, :], axis=-1)
    out_ref[...] = comb


def _tc_combine(idx, d, W1, b1, W2, b2):
    R, K = idx.shape
    return pl.pallas_call(
        _tc_body,
        out_shape=jax.ShapeDtypeStruct((R, K), jnp.float32),
    )(idx, d, W1, b1.reshape(1, -1), W2.reshape(1, -1), b2.reshape(1, 1))


@functools.cache
def _make_sc_scatter(B, S, K, V):
    NC, NS = 2, 16
    NW = NC * NS
    assert B == NW and K % 16 == 0
    CW = 6144
    NCHUNK = V // CW
    TW = V - NCHUNK * CW
    mesh = plsc.VectorSubcoreMesh(core_axis_name="c", subcore_axis_name="s")

    def _scatter_halves(buf, idx_v, val_v, base, width, vals_are_zero):
        for s in range(S):
            srow = jnp.full((16,), s, jnp.int32)
            for h in range(K // 16):
                iv = idx_v[s, pl.ds(h * 16, 16)]
                m = (iv >= base) & (iv < base + width)
                loc = jnp.where(m, iv - base, 0)
                if vals_are_zero:
                    vv = jnp.zeros((16,), jnp.float32)
                else:
                    vv = val_v[s, pl.ds(h * 16, 16)]
                plsc.store_scatter(buf, [srow, loc], vv, mask=m)

    @functools.partial(
        pl.kernel,
        mesh=mesh,
        out_type=jax.ShapeDtypeStruct((B, S, V), jnp.float32),
        compiler_params=pltpu.CompilerParams(needs_layout_passes=False),
        scratch_types=[
            pltpu.VMEM((S, CW), jnp.float32),
            pltpu.VMEM((S, CW), jnp.float32),
            pltpu.VMEM((S, TW), jnp.float32),
            pltpu.VMEM((S, K), jnp.int32),
            pltpu.VMEM((S, K), jnp.float32),
            pltpu.SemaphoreType.DMA,
            pltpu.SemaphoreType.DMA,
            pltpu.SemaphoreType.DMA,
        ],
    )
    def sc_scatter(zeros_hbm, idx_hbm, val_hbm, out_hbm,
                   buf_a, buf_b, tailbuf, idx_v, val_v, sem_a, sem_b, sem_p):
        b = lax.axis_index("s") * NC + lax.axis_index("c")
        bufs = (buf_a, buf_b)
        sems = (sem_a, sem_b)
        pre = [
            pltpu.async_copy(zeros_hbm.at[:, pl.ds(0, CW)], buf_a, sem_p),
            pltpu.async_copy(zeros_hbm.at[:, pl.ds(0, CW)], buf_b, sem_p),
            pltpu.async_copy(zeros_hbm.at[:, pl.ds(CW, TW)], tailbuf, sem_p),
            pltpu.async_copy(idx_hbm.at[pl.ds(b * S, S)], idx_v, sem_p),
            pltpu.async_copy(val_hbm.at[pl.ds(b * S, S)], val_v, sem_p),
        ]
        for cp in pre:
            cp.wait()
        copies = [None, None]
        for c in range(NCHUNK):
            buf = bufs[c % 2]
            if c >= 2:
                copies[c % 2].wait()
                _scatter_halves(buf, idx_v, val_v, (c - 2) * CW, CW, True)
            _scatter_halves(buf, idx_v, val_v, c * CW, CW, False)
            copies[c % 2] = pltpu.async_copy(
                buf, out_hbm.at[b, :, pl.ds(c * CW, CW)], sems[c % 2])
        base = NCHUNK * CW
        _scatter_halves(tailbuf, idx_v, val_v, base, TW, False)
        tail_cp = pltpu.async_copy(
            tailbuf, out_hbm.at[b, :, pl.ds(base, TW)], sem_p)
        copies[0].wait()
        copies[1].wait()
        tail_cp.wait()

    return sc_scatter


def kernel(tgt_index, knn_dists, nmt_prob, W1, b1, W2, b2):
    B, S, K = knn_dists.shape
    V = nmt_prob.shape[-1]
    R = B * S
    idx = tgt_index.reshape(R, K).astype(jnp.int32)
    d = knn_dists.reshape(R, K).astype(jnp.float32)
    vals = _tc_combine(idx, d, W1, b1, W2, b2)
    CW, TW = 6144, V - (V // 6144) * 6144
    zeros_src = jnp.zeros((S, CW + TW), jnp.float32)
    return _make_sc_scatter(B, S, K, V)(zeros_src, idx, vals)
